# Initial kernel scaffold; baseline (speedup 1.0000x reference)
import jax
import jax.numpy as jnp
from jax.experimental import pallas as pl


def kernel(x, edge_index, params):
    raise NotImplementedError("write your pallas kernel here")


# speedup vs baseline: 18.2508x; 18.2508x over previous
import functools

import jax
import jax.numpy as jnp
from jax import lax
from jax.experimental import pallas as pl
from jax.experimental.pallas import tpu as pltpu
from jax.experimental.pallas import tpu_sc as plsc

NN = 10000
NP = 10240
EE = 320000
HEADS = 8
HID = 256
NC = 2
NS = 16
PER_TILE = EE // NS
BB = 80
NB = PER_TILE // BB
ROWS_PT = NP // NS
NEG = -1e30
F32 = jnp.float32
HIGH = jax.lax.Precision.HIGHEST


def _sc_edge(sidx2, dstv, asT2, adT2, hT, m16):
  mesh = plsc.VectorSubcoreMesh(core_axis_name="c", subcore_axis_name="s")

  @functools.partial(
      pl.kernel,
      out_type=(
          jax.ShapeDtypeStruct((NC, NP, 128), F32),
          jax.ShapeDtypeStruct((NP, 16), F32),
      ),
      mesh=mesh,
      compiler_params=pltpu.CompilerParams(use_tc_tiling_on_sc=False),
      scratch_types=[
          pltpu.VMEM((BB,), jnp.int32),
          pltpu.VMEM((BB,), jnp.int32),
          pltpu.VMEM((BB,), jnp.int32),
          pltpu.VMEM((BB, 16), F32),
          pltpu.VMEM((BB, 16), F32),
          pltpu.VMEM((BB, 128), F32),
          pltpu.VMEM((BB, 128), F32),
          pltpu.VMEM((BB, 16), F32),
          pltpu.VMEM((16,), F32),
          pltpu.VMEM_SHARED((NP, 128), F32),
          pltpu.VMEM_SHARED((NP, 16), F32),
      ],
  )
  def k(sidx2_h, dst_h, as_h, ad_h, h_h, m_h, num_h, den_h,
        si_v, si0_v, di_v, as_v, ad_v, h_v, out_v, den_v, m_v,
        num_sh, den_sh):
    c = lax.axis_index("c")
    s = lax.axis_index("s")
    zvec = jnp.zeros((16,), F32)

    @pl.loop(0, BB)
    def _(r):
      for k8 in range(8):
        out_v[r, pl.ds(16 * k8, 16)] = zvec
      den_v[r] = zvec

    r0 = s * ROWS_PT
    for j in range(8):
      pltpu.sync_copy(out_v, num_sh.at[pl.ds(r0 + BB * j, BB)])
      pltpu.sync_copy(den_v, den_sh.at[pl.ds(r0 + BB * j, BB)])
    plsc.subcore_barrier()

    pltpu.sync_copy(m_h, m_v)
    mvec = m_v[...]

    @pl.loop(0, NB)
    def _(b):
      base = s * PER_TILE + b * BB
      pltpu.sync_copy(sidx2_h.at[pl.ds(c * EE + base, BB)], si_v)
      pltpu.sync_copy(sidx2_h.at[pl.ds(base, BB)], si0_v)
      pltpu.sync_copy(dst_h.at[pl.ds(base, BB)], di_v)
      pltpu.sync_copy(as_h.at[si0_v], as_v)
      pltpu.sync_copy(ad_h.at[di_v], ad_v)
      pltpu.sync_copy(h_h.at[si_v], h_v)

      @pl.loop(0, BB)
      def _(e):
        z = as_v[e] + ad_v[e]
        z = jnp.maximum(z, 0.2 * z) - mvec
        w = jnp.exp(z)
        den_v[e] = w
        for kk in range(4):
          iv = jnp.full((16,), 4 * c + kk, jnp.int32)
          bv = w.at[iv].get(mode="promise_in_bounds")
          for t in range(2):
            k2 = 2 * kk + t
            out_v[e, pl.ds(16 * k2, 16)] = h_v[e, pl.ds(16 * k2, 16)] * bv

      pltpu.sync_copy(out_v, num_sh.at[di_v], add=True)

      @pl.when(c == 0)
      def _():
        pltpu.sync_copy(den_v, den_sh.at[di_v], add=True)

    plsc.subcore_barrier()
    pltpu.sync_copy(num_sh.at[pl.ds(r0, ROWS_PT)],
                    num_h.at[c, pl.ds(r0, ROWS_PT)])

    @pl.when(c == 0)
    def _():
      pltpu.sync_copy(den_sh.at[pl.ds(r0, ROWS_PT)],
                      den_h.at[pl.ds(r0, ROWS_PT)])

  return k(sidx2, dstv, asT2, adT2, hT, m16)


def _tc_pre_body(x_ref, w_ref, ps_ref, pd_ref, h2_ref, as2_ref, ad2_ref):
  h = lax.dot(x_ref[...], w_ref[...], precision=HIGH,
              preferred_element_type=F32)
  h2_ref[0] = h[:, :128]
  h2_ref[1] = h[:, 128:]
  lane = lax.broadcasted_iota(jnp.int32, (h.shape[0], 16), 1)
  a_s = lax.dot(h, ps_ref[...], precision=HIGH, preferred_element_type=F32)
  a_s = jnp.where(lane < 8, a_s, NEG)
  as2_ref[0] = a_s
  as2_ref[1] = a_s
  a_d = lax.dot(h, pd_ref[...], precision=HIGH, preferred_element_type=F32)
  a_d = jnp.where(lane < 8, a_d, NEG)
  ad2_ref[0] = a_d
  ad2_ref[1] = a_d


def _tc_pre(x, wg, ps, pd):
  n, d_in = x.shape
  rb = 1000
  return pl.pallas_call(
      _tc_pre_body,
      grid=(n // rb,),
      in_specs=[
          pl.BlockSpec((rb, d_in), lambda i: (i, 0)),
          pl.BlockSpec((d_in, HID), lambda i: (0, 0)),
          pl.BlockSpec((HID, 16), lambda i: (0, 0)),
          pl.BlockSpec((HID, 16), lambda i: (0, 0)),
      ],
      out_specs=[
          pl.BlockSpec((2, rb, 128), lambda i: (0, i, 0)),
          pl.BlockSpec((2, rb, 16), lambda i: (0, i, 0)),
          pl.BlockSpec((2, rb, 16), lambda i: (0, i, 0)),
      ],
      out_shape=[
          jax.ShapeDtypeStruct((2, NP, 128), F32),
          jax.ShapeDtypeStruct((2, NP, 16), F32),
          jax.ShapeDtypeStruct((2, NP, 16), F32),
      ],
  )(x, wg, ps, pd)


def _ln(v, sc, bi):
  mu = jnp.mean(v, axis=1, keepdims=True)
  var = jnp.mean((v - mu) ** 2, axis=1, keepdims=True)
  return (v - mu) / jnp.sqrt(var + 1e-5) * sc + bi


def _tc_post_body(first, num_ref, den_ref, xp_ref, r16_ref, bg_ref,
                  n1s_ref, n1b_ref, w1_ref, b1_ref, w2_ref, b2_ref,
                  n2s_ref, n2b_ref, o_ref):
  den_e = lax.dot(den_ref[...], r16_ref[...], precision=HIGH,
                  preferred_element_type=F32)
  gat = jnp.concatenate([num_ref[0], num_ref[1]], axis=1)
  gat = gat / (den_e + 1e-16) + bg_ref[...]
  if first:
    t = gat
  else:
    t = xp_ref[...] + gat
  t = _ln(t, n1s_ref[...], n1b_ref[...])
  y = jnp.maximum(
      lax.dot(t, w1_ref[...], precision=HIGH, preferred_element_type=F32)
      + b1_ref[...], 0.0)
  y = lax.dot(y, w2_ref[...], precision=HIGH,
              preferred_element_type=F32) + b2_ref[...]
  o_ref[...] = _ln(t + y, n2s_ref[...], n2b_ref[...])


def _tc_post(first, num2, den, xprev, r16, bg, n1s, n1b, w1, b1, w2, b2,
             n2s, n2b):
  n = xprev.shape[0]
  rb = 1000
  dprev = xprev.shape[1]
  return pl.pallas_call(
      functools.partial(_tc_post_body, first),
      grid=(n // rb,),
      in_specs=[
          pl.BlockSpec((2, rb, 128), lambda i: (0, i, 0)),
          pl.BlockSpec((rb, 16), lambda i: (i, 0)),
          pl.BlockSpec((rb, dprev), lambda i: (i, 0)),
          pl.BlockSpec((16, HID), lambda i: (0, 0)),
          pl.BlockSpec((1, HID), lambda i: (0, 0)),
          pl.BlockSpec((1, HID), lambda i: (0, 0)),
          pl.BlockSpec((1, HID), lambda i: (0, 0)),
          pl.BlockSpec((HID, 2 * HID), lambda i: (0, 0)),
          pl.BlockSpec((1, 2 * HID), lambda i: (0, 0)),
          pl.BlockSpec((2 * HID, HID), lambda i: (0, 0)),
          pl.BlockSpec((1, HID), lambda i: (0, 0)),
          pl.BlockSpec((1, HID), lambda i: (0, 0)),
          pl.BlockSpec((1, HID), lambda i: (0, 0)),
      ],
      out_specs=pl.BlockSpec((rb, HID), lambda i: (i, 0)),
      out_shape=jax.ShapeDtypeStruct((n, HID), F32),
  )(num2, den, xprev, r16, bg, n1s, n1b, w1, b1, w2, b2, n2s, n2b)


def kernel(x, edge_index, params):
  src = edge_index[0]
  dst = edge_index[1]
  sidx2 = jnp.concatenate([src, src + NP])

  rows = jnp.arange(HID)
  r16 = (jnp.arange(16)[:, None] == (jnp.arange(HID)[None, :] // 32))
  r16 = r16.astype(F32)

  for li, p in enumerate(params):
    ps = jnp.zeros((HID, 16), F32).at[rows, rows // 32].set(
        p["a_src"].reshape(-1))
    pd = jnp.zeros((HID, 16), F32).at[rows, rows // 32].set(
        p["a_dst"].reshape(-1))

    h2, as2, ad2 = _tc_pre(x, p["W_gat"], ps, pd)

    as_max = jnp.max(as2[0, :, :8], axis=0)
    ad_max = jnp.max(ad2[0, :, :8], axis=0)
    zm = as_max + ad_max
    m8 = jnp.maximum(zm, 0.2 * zm)
    m16 = jnp.concatenate([m8, jnp.zeros((8,), F32)])

    num2, den = _sc_edge(
        sidx2, dst,
        as2[0], ad2[0],
        h2.reshape(2 * NP, 128), m16)

    x = _tc_post(
        li == 0, num2, den, x, r16,
        p["b_gat"].reshape(1, HID),
        p["n1_s"].reshape(1, HID), p["n1_b"].reshape(1, HID),
        p["W1"], p["b1"].reshape(1, 2 * HID),
        p["W2"], p["b2"].reshape(1, HID),
        p["n2_s"].reshape(1, HID), p["n2_b"].reshape(1, HID))
  return x


# speedup vs baseline: 29.4701x; 1.6147x over previous
import functools

import jax
import jax.numpy as jnp
from jax import lax
from jax.experimental import pallas as pl
from jax.experimental.pallas import tpu as pltpu
from jax.experimental.pallas import tpu_sc as plsc

NN = 10000
NP = 10240
EE = 320000
HEADS = 8
HID = 256
NC = 2
NS = 16
PER_TILE = EE // NS
BB = 80
NB = PER_TILE // BB
ROWS_PT = NP // NS
NEG = -1e30
F32 = jnp.float32
HIGH = jax.lax.Precision.HIGHEST


def _sc_edge(sidx2, dstv, asT2, adT2, hT, m16):
  mesh = plsc.VectorSubcoreMesh(core_axis_name="c", subcore_axis_name="s")

  @functools.partial(
      pl.kernel,
      out_type=(
          jax.ShapeDtypeStruct((NC, NP, 128), F32),
          jax.ShapeDtypeStruct((NP, 16), F32),
      ),
      mesh=mesh,
      compiler_params=pltpu.CompilerParams(use_tc_tiling_on_sc=False),
      scratch_types=[
          pltpu.VMEM((BB,), jnp.int32),
          pltpu.VMEM((BB,), jnp.int32),
          pltpu.VMEM((BB,), jnp.int32),
          pltpu.VMEM((BB,), jnp.int32),
          pltpu.VMEM((BB, 16), F32),
          pltpu.VMEM((BB, 16), F32),
          pltpu.VMEM((BB, 16), F32),
          pltpu.VMEM((BB, 16), F32),
          pltpu.VMEM((BB, 128), F32),
          pltpu.VMEM((BB, 128), F32),
          pltpu.VMEM((BB, 128), F32),
          pltpu.VMEM((BB, 16), F32),
          pltpu.VMEM((16,), F32),
          pltpu.VMEM_SHARED((NP, 128), F32),
          pltpu.VMEM_SHARED((NP, 16), F32),
      ] + [pltpu.SemaphoreType.DMA] * 10,
  )
  def k(sidx2_h, dst_h, as_h, ad_h, h_h, m_h, num_h, den_h,
        si_a, si_b, di_a, di_b, as_a, as_b, ad_a, ad_b, h_a, h_b,
        out_v, den_v, m_v, num_sh, den_sh,
        sas_a, sad_a, sh_a, ssi_a, sdi_a, sas_b, sad_b, sh_b, ssi_b, sdi_b):
    c = lax.axis_index("c")
    s = lax.axis_index("s")
    zvec = jnp.zeros((16,), F32)

    @pl.loop(0, BB)
    def _(r):
      for k8 in range(8):
        out_v[r, pl.ds(16 * k8, 16)] = zvec
      den_v[r] = zvec

    r0 = s * ROWS_PT
    for j in range(8):
      pltpu.sync_copy(out_v, num_sh.at[pl.ds(r0 + BB * j, BB)])
      pltpu.sync_copy(den_v, den_sh.at[pl.ds(r0 + BB * j, BB)])
    plsc.subcore_barrier()

    pltpu.sync_copy(m_h, m_v)
    mvec = m_v[...]
    ivs = [jnp.full((16,), 4 * c + kk, jnp.int32) for kk in range(4)]

    seta = (si_a, di_a, as_a, ad_a, h_a, sas_a, sad_a, sh_a, ssi_a, sdi_a)
    setb = (si_b, di_b, as_b, ad_b, h_b, sas_b, sad_b, sh_b, ssi_b, sdi_b)

    def idx_issue(bidx, st):
      base = s * PER_TILE + bidx * BB
      pltpu.async_copy(sidx2_h.at[pl.ds(c * EE + base, BB)], st[0], st[8])
      pltpu.async_copy(dst_h.at[pl.ds(base, BB)], st[1], st[9])

    def idx_wait(st):
      pltpu.make_async_copy(sidx2_h.at[pl.ds(0, BB)], st[0], st[8]).wait()
      pltpu.make_async_copy(dst_h.at[pl.ds(0, BB)], st[1], st[9]).wait()

    def gather_issue(st):
      pltpu.async_copy(as_h.at[st[0]], st[2], st[5])
      pltpu.async_copy(ad_h.at[st[1]], st[3], st[6])
      pltpu.async_copy(h_h.at[st[0]], st[4], st[7])

    def gather_wait(st):
      pltpu.make_async_copy(as_h.at[st[0]], st[2], st[5]).wait()
      pltpu.make_async_copy(ad_h.at[st[1]], st[3], st[6]).wait()
      pltpu.make_async_copy(h_h.at[st[0]], st[4], st[7]).wait()

    def compute_scatter(st):
      asv, adv, hv = st[2], st[3], st[4]

      @pl.loop(0, BB, unroll=4)
      def _(e):
        adrow = adv[e]
        z = asv[e] + adrow
        zm = mvec + adrow
        m = jnp.maximum(zm, 0.2 * zm)
        z = jnp.maximum(z, 0.2 * z) - m
        w = jnp.exp(z)
        den_v[e] = w
        for kk in range(4):
          bv = w.at[ivs[kk]].get(mode="promise_in_bounds")
          for t in range(2):
            k2 = 2 * kk + t
            out_v[e, pl.ds(16 * k2, 16)] = hv[e, pl.ds(16 * k2, 16)] * bv

      pltpu.sync_copy(out_v, num_sh.at[st[1]], add=True)

      @pl.when(c == 0)
      def _():
        pltpu.sync_copy(den_v, den_sh.at[st[1]], add=True)

    idx_issue(0, seta)
    idx_wait(seta)
    gather_issue(seta)
    idx_issue(1, setb)

    @pl.loop(0, (NB - 2) // 2)
    def _(i):
      b0 = 2 * i
      for cur, nxt, boff in ((seta, setb, b0), (setb, seta, b0 + 1)):
        idx_wait(nxt)
        gather_issue(nxt)
        gather_wait(cur)
        compute_scatter(cur)
        idx_issue(boff + 2, cur)

    idx_wait(setb)
    gather_issue(setb)
    gather_wait(seta)
    compute_scatter(seta)
    gather_wait(setb)
    compute_scatter(setb)

    plsc.subcore_barrier()
    pltpu.sync_copy(num_sh.at[pl.ds(r0, ROWS_PT)],
                    num_h.at[c, pl.ds(r0, ROWS_PT)])

    @pl.when(c == 0)
    def _():
      pltpu.sync_copy(den_sh.at[pl.ds(r0, ROWS_PT)],
                      den_h.at[pl.ds(r0, ROWS_PT)])

  return k(sidx2, dstv, asT2, adT2, hT, m16)


def _tc_pre_body(x_ref, w_ref, ps_ref, pd_ref, h2_ref, as2_ref, ad2_ref):
  h = lax.dot(x_ref[...], w_ref[...], preferred_element_type=F32)
  h2_ref[0] = h[:, :128]
  h2_ref[1] = h[:, 128:]
  lane = lax.broadcasted_iota(jnp.int32, (h.shape[0], 16), 1)
  a_s = lax.dot(h, ps_ref[...], precision=HIGH, preferred_element_type=F32)
  a_s = jnp.where(lane < 8, a_s, NEG)
  as2_ref[0] = a_s
  as2_ref[1] = a_s
  a_d = lax.dot(h, pd_ref[...], precision=HIGH, preferred_element_type=F32)
  a_d = jnp.where(lane < 8, a_d, NEG)
  ad2_ref[0] = a_d
  ad2_ref[1] = a_d


def _tc_pre(x, wg, ps, pd):
  n, d_in = x.shape
  rb = 1000
  return pl.pallas_call(
      _tc_pre_body,
      grid=(n // rb,),
      in_specs=[
          pl.BlockSpec((rb, d_in), lambda i: (i, 0)),
          pl.BlockSpec((d_in, HID), lambda i: (0, 0)),
          pl.BlockSpec((HID, 16), lambda i: (0, 0)),
          pl.BlockSpec((HID, 16), lambda i: (0, 0)),
      ],
      out_specs=[
          pl.BlockSpec((2, rb, 128), lambda i: (0, i, 0)),
          pl.BlockSpec((2, rb, 16), lambda i: (0, i, 0)),
          pl.BlockSpec((2, rb, 16), lambda i: (0, i, 0)),
      ],
      out_shape=[
          jax.ShapeDtypeStruct((2, NP, 128), F32),
          jax.ShapeDtypeStruct((2, NP, 16), F32),
          jax.ShapeDtypeStruct((2, NP, 16), F32),
      ],
  )(x, wg, ps, pd)


def _ln(v, sc, bi):
  mu = jnp.mean(v, axis=1, keepdims=True)
  var = jnp.mean((v - mu) ** 2, axis=1, keepdims=True)
  return (v - mu) / jnp.sqrt(var + 1e-5) * sc + bi


def _tc_post_body(first, num_ref, den_ref, xp_ref, r16_ref, bg_ref,
                  n1s_ref, n1b_ref, w1_ref, b1_ref, w2_ref, b2_ref,
                  n2s_ref, n2b_ref, o_ref):
  den_e = lax.dot(den_ref[...], r16_ref[...], precision=HIGH,
                  preferred_element_type=F32)
  gat = jnp.concatenate([num_ref[0], num_ref[1]], axis=1)
  gat = jnp.where(den_e > 0.0, gat / den_e, 0.0) + bg_ref[...]
  if first:
    t = gat
  else:
    t = xp_ref[...] + gat
  t = _ln(t, n1s_ref[...], n1b_ref[...])
  y = jnp.maximum(
      lax.dot(t, w1_ref[...], preferred_element_type=F32)
      + b1_ref[...], 0.0)
  y = lax.dot(y, w2_ref[...], preferred_element_type=F32) + b2_ref[...]
  o_ref[...] = _ln(t + y, n2s_ref[...], n2b_ref[...])


def _tc_post(first, num2, den, xprev, r16, bg, n1s, n1b, w1, b1, w2, b2,
             n2s, n2b):
  n = xprev.shape[0]
  rb = 1000
  dprev = xprev.shape[1]
  return pl.pallas_call(
      functools.partial(_tc_post_body, first),
      grid=(n // rb,),
      in_specs=[
          pl.BlockSpec((2, rb, 128), lambda i: (0, i, 0)),
          pl.BlockSpec((rb, 16), lambda i: (i, 0)),
          pl.BlockSpec((rb, dprev), lambda i: (i, 0)),
          pl.BlockSpec((16, HID), lambda i: (0, 0)),
          pl.BlockSpec((1, HID), lambda i: (0, 0)),
          pl.BlockSpec((1, HID), lambda i: (0, 0)),
          pl.BlockSpec((1, HID), lambda i: (0, 0)),
          pl.BlockSpec((HID, 2 * HID), lambda i: (0, 0)),
          pl.BlockSpec((1, 2 * HID), lambda i: (0, 0)),
          pl.BlockSpec((2 * HID, HID), lambda i: (0, 0)),
          pl.BlockSpec((1, HID), lambda i: (0, 0)),
          pl.BlockSpec((1, HID), lambda i: (0, 0)),
          pl.BlockSpec((1, HID), lambda i: (0, 0)),
      ],
      out_specs=pl.BlockSpec((rb, HID), lambda i: (i, 0)),
      out_shape=jax.ShapeDtypeStruct((n, HID), F32),
  )(num2, den, xprev, r16, bg, n1s, n1b, w1, b1, w2, b2, n2s, n2b)


def kernel(x, edge_index, params):
  src = edge_index[0]
  dst = edge_index[1]
  sidx2 = jnp.concatenate([src, src + NP])

  rows = jnp.arange(HID)
  r16 = (jnp.arange(16)[:, None] == (jnp.arange(HID)[None, :] // 32))
  r16 = r16.astype(F32)

  for li, p in enumerate(params):
    ps = jnp.zeros((HID, 16), F32).at[rows, rows // 32].set(
        p["a_src"].reshape(-1))
    pd = jnp.zeros((HID, 16), F32).at[rows, rows // 32].set(
        p["a_dst"].reshape(-1))

    h2, as2, ad2 = _tc_pre(x, p["W_gat"], ps, pd)

    as_max = jnp.max(as2[0, :, :8], axis=0)
    m16 = jnp.concatenate([as_max, jnp.zeros((8,), F32)])

    num2, den = _sc_edge(
        sidx2, dst,
        as2.reshape(2 * NP, 16), ad2[0],
        h2.reshape(2 * NP, 128), m16)

    x = _tc_post(
        li == 0, num2, den, x, r16,
        p["b_gat"].reshape(1, HID),
        p["n1_s"].reshape(1, HID), p["n1_b"].reshape(1, HID),
        p["W1"], p["b1"].reshape(1, 2 * HID),
        p["W2"], p["b2"].reshape(1, HID),
        p["n2_s"].reshape(1, HID), p["n2_b"].reshape(1, HID))
  return x


# speedup vs baseline: 82.5563x; 2.8014x over previous
import functools

import jax
import jax.numpy as jnp
from jax import lax
from jax.experimental import pallas as pl
from jax.experimental.pallas import tpu as pltpu
from jax.experimental.pallas import tpu_sc as plsc

NN = 10000
NP = 10240
EE = 320000
HEADS = 8
HID = 256
NC = 2
NS = 16
PER_TILE = EE // NS
BB = 80
NB = PER_TILE // BB
ROWS_PT = NP // NS
NEG = -1e30
F32 = jnp.float32
HIGH = jax.lax.Precision.HIGHEST


def _sc_edge(sidx2, dstv, asT2, adT2, hT, m16):
  mesh = plsc.VectorSubcoreMesh(core_axis_name="c", subcore_axis_name="s")

  @functools.partial(
      pl.kernel,
      out_type=(
          jax.ShapeDtypeStruct((NC, NP, 128), F32),
          jax.ShapeDtypeStruct((NP, 16), F32),
      ),
      mesh=mesh,
      compiler_params=pltpu.CompilerParams(use_tc_tiling_on_sc=False),
      scratch_types=[
          pltpu.VMEM((BB,), jnp.int32),
          pltpu.VMEM((BB,), jnp.int32),
          pltpu.VMEM((BB,), jnp.int32),
          pltpu.VMEM((BB,), jnp.int32),
          pltpu.VMEM((BB, 16), F32),
          pltpu.VMEM((BB, 16), F32),
          pltpu.VMEM((BB, 16), F32),
          pltpu.VMEM((BB, 16), F32),
          pltpu.VMEM((BB, 128), F32),
          pltpu.VMEM((BB, 128), F32),
          pltpu.VMEM((BB, 128), F32),
          pltpu.VMEM((BB, 16), F32),
          pltpu.VMEM((16,), F32),
          pltpu.VMEM_SHARED((NP, 128), F32),
          pltpu.VMEM_SHARED((NP, 16), F32),
      ] + [pltpu.SemaphoreType.DMA] * 10,
  )
  def k(sidx2_h, dst_h, as_h, ad_h, h_h, m_h, num_h, den_h,
        si_a, si_b, di_a, di_b, as_a, as_b, ad_a, ad_b, h_a, h_b,
        out_v, den_v, m_v, num_sh, den_sh,
        sas_a, sad_a, sh_a, ssi_a, sdi_a, sas_b, sad_b, sh_b, ssi_b, sdi_b):
    c = lax.axis_index("c")
    s = lax.axis_index("s")
    zvec = jnp.zeros((16,), F32)

    @pl.loop(0, BB)
    def _(r):
      for k8 in range(8):
        out_v[r, pl.ds(16 * k8, 16)] = zvec
      den_v[r] = zvec

    r0 = s * ROWS_PT
    for j in range(8):
      pltpu.sync_copy(out_v, num_sh.at[pl.ds(r0 + BB * j, BB)])
      pltpu.sync_copy(den_v, den_sh.at[pl.ds(r0 + BB * j, BB)])
    plsc.subcore_barrier()

    pltpu.sync_copy(m_h, m_v)
    mvec = m_v[...]
    ivs = [jnp.full((16,), 4 * c + kk, jnp.int32) for kk in range(4)]

    seta = (si_a, di_a, as_a, ad_a, h_a, sas_a, sad_a, sh_a, ssi_a, sdi_a)
    setb = (si_b, di_b, as_b, ad_b, h_b, sas_b, sad_b, sh_b, ssi_b, sdi_b)

    def idx_issue(bidx, st):
      base = s * PER_TILE + bidx * BB
      pltpu.async_copy(sidx2_h.at[pl.ds(c * EE + base, BB)], st[0], st[8])
      pltpu.async_copy(dst_h.at[pl.ds(base, BB)], st[1], st[9])

    def idx_wait(st):
      pltpu.make_async_copy(sidx2_h.at[pl.ds(0, BB)], st[0], st[8]).wait()
      pltpu.make_async_copy(dst_h.at[pl.ds(0, BB)], st[1], st[9]).wait()

    def gather_issue(st):
      pltpu.async_copy(as_h.at[st[0]], st[2], st[5])
      pltpu.async_copy(ad_h.at[st[1]], st[3], st[6])
      pltpu.async_copy(h_h.at[st[0]], st[4], st[7])

    def gather_wait(st):
      pltpu.make_async_copy(as_h.at[st[0]], st[2], st[5]).wait()
      pltpu.make_async_copy(ad_h.at[st[1]], st[3], st[6]).wait()
      pltpu.make_async_copy(h_h.at[st[0]], st[4], st[7]).wait()

    def compute_scatter(st):
      asv, adv, hv = st[2], st[3], st[4]

      @plsc.parallel_loop(0, BB, unroll=4)
      def _(e):
        adrow = adv[e]
        z = asv[e] + adrow
        zm = mvec + adrow
        m = jnp.maximum(zm, 0.2 * zm)
        z = jnp.maximum(z, 0.2 * z) - m
        w = jnp.exp(z)
        den_v[e] = w
        for kk in range(4):
          bv = w.at[ivs[kk]].get(mode="promise_in_bounds")
          for t in range(2):
            k2 = 2 * kk + t
            out_v[e, pl.ds(16 * k2, 16)] = hv[e, pl.ds(16 * k2, 16)] * bv

      pltpu.sync_copy(out_v, num_sh.at[st[1]], add=True)

      @pl.when(c == 0)
      def _():
        pltpu.sync_copy(den_v, den_sh.at[st[1]], add=True)

    idx_issue(0, seta)
    idx_wait(seta)
    gather_issue(seta)
    idx_issue(1, setb)

    @pl.loop(0, (NB - 2) // 2)
    def _(i):
      b0 = 2 * i
      for cur, nxt, boff in ((seta, setb, b0), (setb, seta, b0 + 1)):
        idx_wait(nxt)
        gather_issue(nxt)
        gather_wait(cur)
        compute_scatter(cur)
        idx_issue(boff + 2, cur)

    idx_wait(setb)
    gather_issue(setb)
    gather_wait(seta)
    compute_scatter(seta)
    gather_wait(setb)
    compute_scatter(setb)

    plsc.subcore_barrier()
    pltpu.sync_copy(num_sh.at[pl.ds(r0, ROWS_PT)],
                    num_h.at[c, pl.ds(r0, ROWS_PT)])

    @pl.when(c == 0)
    def _():
      pltpu.sync_copy(den_sh.at[pl.ds(r0, ROWS_PT)],
                      den_h.at[pl.ds(r0, ROWS_PT)])

  return k(sidx2, dstv, asT2, adT2, hT, m16)


def _tc_pre_body(x_ref, w_ref, ps_ref, pd_ref, h2_ref, as2_ref, ad2_ref):
  h = lax.dot(x_ref[...], w_ref[...], preferred_element_type=F32)
  h2_ref[0] = h[:, :128]
  h2_ref[1] = h[:, 128:]
  lane = lax.broadcasted_iota(jnp.int32, (h.shape[0], 16), 1)
  a_s = lax.dot(h, ps_ref[...], precision=HIGH, preferred_element_type=F32)
  a_s = jnp.where(lane < 8, a_s, NEG)
  as2_ref[0] = a_s
  as2_ref[1] = a_s
  a_d = lax.dot(h, pd_ref[...], precision=HIGH, preferred_element_type=F32)
  a_d = jnp.where(lane < 8, a_d, NEG)
  ad2_ref[0] = a_d
  ad2_ref[1] = a_d


def _tc_pre(x, wg, ps, pd):
  n, d_in = x.shape
  rb = 1000
  return pl.pallas_call(
      _tc_pre_body,
      grid=(n // rb,),
      in_specs=[
          pl.BlockSpec((rb, d_in), lambda i: (i, 0)),
          pl.BlockSpec((d_in, HID), lambda i: (0, 0)),
          pl.BlockSpec((HID, 16), lambda i: (0, 0)),
          pl.BlockSpec((HID, 16), lambda i: (0, 0)),
      ],
      out_specs=[
          pl.BlockSpec((2, rb, 128), lambda i: (0, i, 0)),
          pl.BlockSpec((2, rb, 16), lambda i: (0, i, 0)),
          pl.BlockSpec((2, rb, 16), lambda i: (0, i, 0)),
      ],
      out_shape=[
          jax.ShapeDtypeStruct((2, NP, 128), F32),
          jax.ShapeDtypeStruct((2, NP, 16), F32),
          jax.ShapeDtypeStruct((2, NP, 16), F32),
      ],
  )(x, wg, ps, pd)


def _ln(v, sc, bi):
  mu = jnp.mean(v, axis=1, keepdims=True)
  var = jnp.mean((v - mu) ** 2, axis=1, keepdims=True)
  return (v - mu) / jnp.sqrt(var + 1e-5) * sc + bi


def _tc_post_body(first, num_ref, den_ref, xp_ref, r16_ref, bg_ref,
                  n1s_ref, n1b_ref, w1_ref, b1_ref, w2_ref, b2_ref,
                  n2s_ref, n2b_ref, o_ref):
  den_e = lax.dot(den_ref[...], r16_ref[...], precision=HIGH,
                  preferred_element_type=F32)
  gat = jnp.concatenate([num_ref[0], num_ref[1]], axis=1)
  gat = jnp.where(den_e > 0.0, gat / den_e, 0.0) + bg_ref[...]
  if first:
    t = gat
  else:
    t = xp_ref[...] + gat
  t = _ln(t, n1s_ref[...], n1b_ref[...])
  y = jnp.maximum(
      lax.dot(t, w1_ref[...], preferred_element_type=F32)
      + b1_ref[...], 0.0)
  y = lax.dot(y, w2_ref[...], preferred_element_type=F32) + b2_ref[...]
  o_ref[...] = _ln(t + y, n2s_ref[...], n2b_ref[...])


def _tc_post(first, num2, den, xprev, r16, bg, n1s, n1b, w1, b1, w2, b2,
             n2s, n2b):
  n = xprev.shape[0]
  rb = 1000
  dprev = xprev.shape[1]
  return pl.pallas_call(
      functools.partial(_tc_post_body, first),
      grid=(n // rb,),
      in_specs=[
          pl.BlockSpec((2, rb, 128), lambda i: (0, i, 0)),
          pl.BlockSpec((rb, 16), lambda i: (i, 0)),
          pl.BlockSpec((rb, dprev), lambda i: (i, 0)),
          pl.BlockSpec((16, HID), lambda i: (0, 0)),
          pl.BlockSpec((1, HID), lambda i: (0, 0)),
          pl.BlockSpec((1, HID), lambda i: (0, 0)),
          pl.BlockSpec((1, HID), lambda i: (0, 0)),
          pl.BlockSpec((HID, 2 * HID), lambda i: (0, 0)),
          pl.BlockSpec((1, 2 * HID), lambda i: (0, 0)),
          pl.BlockSpec((2 * HID, HID), lambda i: (0, 0)),
          pl.BlockSpec((1, HID), lambda i: (0, 0)),
          pl.BlockSpec((1, HID), lambda i: (0, 0)),
          pl.BlockSpec((1, HID), lambda i: (0, 0)),
      ],
      out_specs=pl.BlockSpec((rb, HID), lambda i: (i, 0)),
      out_shape=jax.ShapeDtypeStruct((n, HID), F32),
  )(num2, den, xprev, r16, bg, n1s, n1b, w1, b1, w2, b2, n2s, n2b)


def kernel(x, edge_index, params):
  src = edge_index[0]
  dst = edge_index[1]
  sidx2 = jnp.concatenate([src, src + NP])

  rows = jnp.arange(HID)
  r16 = (jnp.arange(16)[:, None] == (jnp.arange(HID)[None, :] // 32))
  r16 = r16.astype(F32)

  for li, p in enumerate(params):
    ps = jnp.zeros((HID, 16), F32).at[rows, rows // 32].set(
        p["a_src"].reshape(-1))
    pd = jnp.zeros((HID, 16), F32).at[rows, rows // 32].set(
        p["a_dst"].reshape(-1))

    h2, as2, ad2 = _tc_pre(x, p["W_gat"], ps, pd)

    as_max = jnp.max(as2[0, :, :8], axis=0)
    m16 = jnp.concatenate([as_max, jnp.zeros((8,), F32)])

    num2, den = _sc_edge(
        sidx2, dst,
        as2.reshape(2 * NP, 16), ad2[0],
        h2.reshape(2 * NP, 128), m16)

    x = _tc_post(
        li == 0, num2, den, x, r16,
        p["b_gat"].reshape(1, HID),
        p["n1_s"].reshape(1, HID), p["n1_b"].reshape(1, HID),
        p["W1"], p["b1"].reshape(1, 2 * HID),
        p["W2"], p["b2"].reshape(1, HID),
        p["n2_s"].reshape(1, HID), p["n2_b"].reshape(1, HID))
  return x


# speedup vs baseline: 97.6345x; 1.1826x over previous
import functools

import jax
import jax.numpy as jnp
from jax import lax
from jax.experimental import pallas as pl
from jax.experimental.pallas import tpu as pltpu
from jax.experimental.pallas import tpu_sc as plsc

NN = 10000
NP = 10240
EE = 320000
HEADS = 8
HID = 256
NC = 2
NS = 16
PER_TILE = EE // NS
BB = 80
NB = PER_TILE // BB
ROWS_PT = NP // NS
NEG = -1e30
F32 = jnp.float32
HIGH = jax.lax.Precision.HIGHEST


def _sc_edge(sidx2, dstv, asT2, adT2, hT, m16):
  mesh = plsc.VectorSubcoreMesh(core_axis_name="c", subcore_axis_name="s")

  @functools.partial(
      pl.kernel,
      out_type=(
          jax.ShapeDtypeStruct((NC, NP, 128), F32),
          jax.ShapeDtypeStruct((NP, 16), F32),
      ),
      mesh=mesh,
      compiler_params=pltpu.CompilerParams(use_tc_tiling_on_sc=False),
      scratch_types=[
          pltpu.VMEM((BB,), jnp.int32),
          pltpu.VMEM((BB,), jnp.int32),
          pltpu.VMEM((BB,), jnp.int32),
          pltpu.VMEM((BB,), jnp.int32),
          pltpu.VMEM((BB, 16), F32),
          pltpu.VMEM((BB, 16), F32),
          pltpu.VMEM((BB, 16), F32),
          pltpu.VMEM((BB, 16), F32),
          pltpu.VMEM((BB, 128), F32),
          pltpu.VMEM((BB, 128), F32),
          pltpu.VMEM((BB, 128), F32),
          pltpu.VMEM((BB, 16), F32),
          pltpu.VMEM((16,), F32),
          pltpu.VMEM_SHARED((NP, 128), F32),
          pltpu.VMEM_SHARED((NP, 16), F32),
      ] + [pltpu.SemaphoreType.DMA] * 12,
  )
  def k(sidx2_h, dst_h, as_h, ad_h, h_h, m_h, num_h, den_h,
        si_a, si_b, di_a, di_b, as_a, as_b, ad_a, ad_b, h_a, h_b,
        out_v, den_v, m_v, num_sh, den_sh,
        sas_a, sad_a, sh_a, ssi_a, sdi_a, sas_b, sad_b, sh_b, ssi_b, sdi_b,
        sems, semsd):
    c = lax.axis_index("c")
    s = lax.axis_index("s")
    zvec = jnp.zeros((16,), F32)

    @pl.loop(0, BB)
    def _(r):
      for k8 in range(8):
        out_v[r, pl.ds(16 * k8, 16)] = zvec
      den_v[r] = zvec

    r0 = s * ROWS_PT
    for j in range(8):
      pltpu.sync_copy(out_v, num_sh.at[pl.ds(r0 + BB * j, BB)])
      pltpu.sync_copy(den_v, den_sh.at[pl.ds(r0 + BB * j, BB)])
    izvec = jnp.zeros((16,), jnp.int32)
    for j in range(5):
      di_a[pl.ds(16 * j, 16)] = izvec
    pltpu.async_copy(out_v, num_sh.at[di_a], sems, add=True)

    @pl.when(c == 0)
    def _():
      pltpu.async_copy(den_v, den_sh.at[di_a], semsd, add=True)
    plsc.subcore_barrier()

    pltpu.sync_copy(m_h, m_v)
    mvec = m_v[...]
    ivs = [jnp.full((16,), 4 * c + kk, jnp.int32) for kk in range(4)]

    seta = (si_a, di_a, as_a, ad_a, h_a, sas_a, sad_a, sh_a, ssi_a, sdi_a)
    setb = (si_b, di_b, as_b, ad_b, h_b, sas_b, sad_b, sh_b, ssi_b, sdi_b)

    def idx_issue(bidx, st):
      base = s * PER_TILE + bidx * BB
      pltpu.async_copy(sidx2_h.at[pl.ds(c * EE + base, BB)], st[0], st[8])
      pltpu.async_copy(dst_h.at[pl.ds(base, BB)], st[1], st[9])

    def idx_wait(st):
      pltpu.make_async_copy(sidx2_h.at[pl.ds(0, BB)], st[0], st[8]).wait()
      pltpu.make_async_copy(dst_h.at[pl.ds(0, BB)], st[1], st[9]).wait()

    def gather_issue(st):
      pltpu.async_copy(as_h.at[st[0]], st[2], st[5])
      pltpu.async_copy(ad_h.at[st[1]], st[3], st[6])
      pltpu.async_copy(h_h.at[st[0]], st[4], st[7])

    def gather_wait(st):
      pltpu.make_async_copy(as_h.at[st[0]], st[2], st[5]).wait()
      pltpu.make_async_copy(ad_h.at[st[1]], st[3], st[6]).wait()
      pltpu.make_async_copy(h_h.at[st[0]], st[4], st[7]).wait()

    def compute_scatter(st):
      asv, adv, hv = st[2], st[3], st[4]
      pltpu.make_async_copy(out_v, num_sh.at[st[1]], sems).wait()

      @pl.when(c == 0)
      def _():
        pltpu.make_async_copy(den_v, den_sh.at[st[1]], semsd).wait()

      @plsc.parallel_loop(0, BB, unroll=8)
      def _(e):
        adrow = adv[e]
        z = asv[e] + adrow
        zm = mvec + adrow
        m = jnp.maximum(zm, 0.2 * zm)
        z = jnp.maximum(z, 0.2 * z) - m
        w = jnp.exp(z)
        den_v[e] = w
        for kk in range(4):
          bv = w.at[ivs[kk]].get(mode="promise_in_bounds")
          for t in range(2):
            k2 = 2 * kk + t
            out_v[e, pl.ds(16 * k2, 16)] = hv[e, pl.ds(16 * k2, 16)] * bv

      pltpu.async_copy(out_v, num_sh.at[st[1]], sems, add=True)

      @pl.when(c == 0)
      def _():
        pltpu.async_copy(den_v, den_sh.at[st[1]], semsd, add=True)

    idx_issue(0, seta)
    idx_wait(seta)
    gather_issue(seta)
    idx_issue(1, setb)

    @pl.loop(0, (NB - 2) // 2)
    def _(i):
      b0 = 2 * i
      for cur, nxt, boff in ((seta, setb, b0), (setb, seta, b0 + 1)):
        idx_wait(nxt)
        gather_issue(nxt)
        gather_wait(cur)
        compute_scatter(cur)
        idx_issue(boff + 2, cur)

    idx_wait(setb)
    gather_issue(setb)
    gather_wait(seta)
    compute_scatter(seta)
    gather_wait(setb)
    compute_scatter(setb)
    pltpu.make_async_copy(out_v, num_sh.at[setb[1]], sems).wait()

    @pl.when(c == 0)
    def _():
      pltpu.make_async_copy(den_v, den_sh.at[setb[1]], semsd).wait()

    plsc.subcore_barrier()
    pltpu.sync_copy(num_sh.at[pl.ds(r0, ROWS_PT)],
                    num_h.at[c, pl.ds(r0, ROWS_PT)])

    @pl.when(c == 0)
    def _():
      pltpu.sync_copy(den_sh.at[pl.ds(r0, ROWS_PT)],
                      den_h.at[pl.ds(r0, ROWS_PT)])

  return k(sidx2, dstv, asT2, adT2, hT, m16)


def _tc_pre_body(x_ref, w_ref, ps_ref, pd_ref, h2_ref, as2_ref, ad2_ref):
  h = lax.dot(x_ref[...], w_ref[...], preferred_element_type=F32)
  h2_ref[0] = h[:, :128]
  h2_ref[1] = h[:, 128:]
  lane = lax.broadcasted_iota(jnp.int32, (h.shape[0], 16), 1)
  a_s = lax.dot(h, ps_ref[...], precision=HIGH, preferred_element_type=F32)
  a_s = jnp.where(lane < 8, a_s, NEG)
  as2_ref[0] = a_s
  as2_ref[1] = a_s
  a_d = lax.dot(h, pd_ref[...], precision=HIGH, preferred_element_type=F32)
  a_d = jnp.where(lane < 8, a_d, NEG)
  ad2_ref[0] = a_d
  ad2_ref[1] = a_d


def _tc_pre(x, wg, ps, pd):
  n, d_in = x.shape
  rb = 1000
  return pl.pallas_call(
      _tc_pre_body,
      grid=(n // rb,),
      in_specs=[
          pl.BlockSpec((rb, d_in), lambda i: (i, 0)),
          pl.BlockSpec((d_in, HID), lambda i: (0, 0)),
          pl.BlockSpec((HID, 16), lambda i: (0, 0)),
          pl.BlockSpec((HID, 16), lambda i: (0, 0)),
      ],
      out_specs=[
          pl.BlockSpec((2, rb, 128), lambda i: (0, i, 0)),
          pl.BlockSpec((2, rb, 16), lambda i: (0, i, 0)),
          pl.BlockSpec((2, rb, 16), lambda i: (0, i, 0)),
      ],
      out_shape=[
          jax.ShapeDtypeStruct((2, NP, 128), F32),
          jax.ShapeDtypeStruct((2, NP, 16), F32),
          jax.ShapeDtypeStruct((2, NP, 16), F32),
      ],
  )(x, wg, ps, pd)


def _ln(v, sc, bi):
  mu = jnp.mean(v, axis=1, keepdims=True)
  var = jnp.mean((v - mu) ** 2, axis=1, keepdims=True)
  return (v - mu) / jnp.sqrt(var + 1e-5) * sc + bi


def _tc_post_body(first, num_ref, den_ref, xp_ref, r16_ref, bg_ref,
                  n1s_ref, n1b_ref, w1_ref, b1_ref, w2_ref, b2_ref,
                  n2s_ref, n2b_ref, o_ref):
  den_e = lax.dot(den_ref[...], r16_ref[...], precision=HIGH,
                  preferred_element_type=F32)
  gat = jnp.concatenate([num_ref[0], num_ref[1]], axis=1)
  gat = jnp.where(den_e > 0.0, gat / den_e, 0.0) + bg_ref[...]
  if first:
    t = gat
  else:
    t = xp_ref[...] + gat
  t = _ln(t, n1s_ref[...], n1b_ref[...])
  y = jnp.maximum(
      lax.dot(t, w1_ref[...], preferred_element_type=F32)
      + b1_ref[...], 0.0)
  y = lax.dot(y, w2_ref[...], preferred_element_type=F32) + b2_ref[...]
  o_ref[...] = _ln(t + y, n2s_ref[...], n2b_ref[...])


def _tc_post(first, num2, den, xprev, r16, bg, n1s, n1b, w1, b1, w2, b2,
             n2s, n2b):
  n = xprev.shape[0]
  rb = 1000
  dprev = xprev.shape[1]
  return pl.pallas_call(
      functools.partial(_tc_post_body, first),
      grid=(n // rb,),
      in_specs=[
          pl.BlockSpec((2, rb, 128), lambda i: (0, i, 0)),
          pl.BlockSpec((rb, 16), lambda i: (i, 0)),
          pl.BlockSpec((rb, dprev), lambda i: (i, 0)),
          pl.BlockSpec((16, HID), lambda i: (0, 0)),
          pl.BlockSpec((1, HID), lambda i: (0, 0)),
          pl.BlockSpec((1, HID), lambda i: (0, 0)),
          pl.BlockSpec((1, HID), lambda i: (0, 0)),
          pl.BlockSpec((HID, 2 * HID), lambda i: (0, 0)),
          pl.BlockSpec((1, 2 * HID), lambda i: (0, 0)),
          pl.BlockSpec((2 * HID, HID), lambda i: (0, 0)),
          pl.BlockSpec((1, HID), lambda i: (0, 0)),
          pl.BlockSpec((1, HID), lambda i: (0, 0)),
          pl.BlockSpec((1, HID), lambda i: (0, 0)),
      ],
      out_specs=pl.BlockSpec((rb, HID), lambda i: (i, 0)),
      out_shape=jax.ShapeDtypeStruct((n, HID), F32),
  )(num2, den, xprev, r16, bg, n1s, n1b, w1, b1, w2, b2, n2s, n2b)


def kernel(x, edge_index, params):
  src = edge_index[0]
  dst = edge_index[1]
  sidx2 = jnp.concatenate([src, src + NP])

  rows = jnp.arange(HID)
  r16 = (jnp.arange(16)[:, None] == (jnp.arange(HID)[None, :] // 32))
  r16 = r16.astype(F32)

  for li, p in enumerate(params):
    ps = jnp.zeros((HID, 16), F32).at[rows, rows // 32].set(
        p["a_src"].reshape(-1))
    pd = jnp.zeros((HID, 16), F32).at[rows, rows // 32].set(
        p["a_dst"].reshape(-1))

    h2, as2, ad2 = _tc_pre(x, p["W_gat"], ps, pd)

    as_max = jnp.max(as2[0, :, :8], axis=0)
    m16 = jnp.concatenate([as_max, jnp.zeros((8,), F32)])

    num2, den = _sc_edge(
        sidx2, dst,
        as2.reshape(2 * NP, 16), ad2[0],
        h2.reshape(2 * NP, 128), m16)

    x = _tc_post(
        li == 0, num2, den, x, r16,
        p["b_gat"].reshape(1, HID),
        p["n1_s"].reshape(1, HID), p["n1_b"].reshape(1, HID),
        p["W1"], p["b1"].reshape(1, 2 * HID),
        p["W2"], p["b2"].reshape(1, HID),
        p["n2_s"].reshape(1, HID), p["n2_b"].reshape(1, HID))
  return x


# speedup vs baseline: 98.0928x; 1.0047x over previous
import functools

import jax
import jax.numpy as jnp
from jax import lax
from jax.experimental import pallas as pl
from jax.experimental.pallas import tpu as pltpu
from jax.experimental.pallas import tpu_sc as plsc

NN = 10000
NP = 10240
EE = 320000
HEADS = 8
HID = 256
NC = 2
NS = 16
PER_TILE = EE // NS
BB = 80
NB = PER_TILE // BB
ROWS_PT = NP // NS
NEG = -1e30
F32 = jnp.float32
HIGH = jax.lax.Precision.HIGHEST


def _sc_edge(sidx2, dstv, asT2, adT2, hT, m16):
  mesh = plsc.VectorSubcoreMesh(core_axis_name="c", subcore_axis_name="s")

  @functools.partial(
      pl.kernel,
      out_type=(
          jax.ShapeDtypeStruct((NC, NP, 128), F32),
          jax.ShapeDtypeStruct((NP, 16), F32),
      ),
      mesh=mesh,
      compiler_params=pltpu.CompilerParams(use_tc_tiling_on_sc=False),
      scratch_types=[
          pltpu.VMEM((BB,), jnp.int32),
          pltpu.VMEM((BB,), jnp.int32),
          pltpu.VMEM((BB,), jnp.int32),
          pltpu.VMEM((BB,), jnp.int32),
          pltpu.VMEM((BB, 16), F32),
          pltpu.VMEM((BB, 16), F32),
          pltpu.VMEM((BB, 16), F32),
          pltpu.VMEM((BB, 16), F32),
          pltpu.VMEM((BB, 128), F32),
          pltpu.VMEM((BB, 128), F32),
          pltpu.VMEM((BB, 128), F32),
          pltpu.VMEM((BB, 16), F32),
          pltpu.VMEM((16,), F32),
          pltpu.VMEM_SHARED((NP, 128), F32),
          pltpu.VMEM_SHARED((NP, 16), F32),
      ] + [pltpu.SemaphoreType.DMA] * 12,
  )
  def k(sidx2_h, dst_h, as_h, ad_h, h_h, m_h, num_h, den_h,
        si_a, si_b, di_a, di_b, as_a, as_b, ad_a, ad_b, h_a, h_b,
        out_v, den_v, m_v, num_sh, den_sh,
        sas_a, sad_a, sh_a, ssi_a, sdi_a, sas_b, sad_b, sh_b, ssi_b, sdi_b,
        sems, semsd):
    c = lax.axis_index("c")
    s = lax.axis_index("s")
    zvec = jnp.zeros((16,), F32)

    @pl.loop(0, BB)
    def _(r):
      for k8 in range(8):
        out_v[r, pl.ds(16 * k8, 16)] = zvec
      den_v[r] = zvec

    r0 = s * ROWS_PT
    for j in range(8):
      pltpu.sync_copy(out_v, num_sh.at[pl.ds(r0 + BB * j, BB)])
      pltpu.sync_copy(den_v, den_sh.at[pl.ds(r0 + BB * j, BB)])
    izvec = jnp.zeros((16,), jnp.int32)
    for j in range(5):
      di_a[pl.ds(16 * j, 16)] = izvec
    pltpu.async_copy(out_v, num_sh.at[di_a], sems, add=True)

    @pl.when(c == 0)
    def _():
      pltpu.async_copy(den_v, den_sh.at[di_a], semsd, add=True)
    plsc.subcore_barrier()

    pltpu.sync_copy(m_h, m_v)
    mvec = m_v[...]
    ivs = [jnp.full((16,), 4 * c + kk, jnp.int32) for kk in range(4)]

    seta = (si_a, di_a, as_a, ad_a, h_a, sas_a, sad_a, sh_a, ssi_a, sdi_a)
    setb = (si_b, di_b, as_b, ad_b, h_b, sas_b, sad_b, sh_b, ssi_b, sdi_b)

    def idx_issue(bidx, st):
      base = s * PER_TILE + bidx * BB
      pltpu.async_copy(sidx2_h.at[pl.ds(c * EE + base, BB)], st[0], st[8])
      pltpu.async_copy(dst_h.at[pl.ds(base, BB)], st[1], st[9])

    def idx_wait(st):
      pltpu.make_async_copy(sidx2_h.at[pl.ds(0, BB)], st[0], st[8]).wait()
      pltpu.make_async_copy(dst_h.at[pl.ds(0, BB)], st[1], st[9]).wait()

    def gather_issue(st):
      pltpu.async_copy(as_h.at[st[0]], st[2], st[5])
      pltpu.async_copy(ad_h.at[st[1]], st[3], st[6])
      pltpu.async_copy(h_h.at[st[0]], st[4], st[7])

    def gather_wait(st):
      pltpu.make_async_copy(as_h.at[st[0]], st[2], st[5]).wait()
      pltpu.make_async_copy(ad_h.at[st[1]], st[3], st[6]).wait()
      pltpu.make_async_copy(h_h.at[st[0]], st[4], st[7]).wait()

    def compute_scatter(st):
      asv, adv, hv = st[2], st[3], st[4]
      pltpu.make_async_copy(out_v, num_sh.at[st[1]], sems).wait()

      @pl.when(c == 0)
      def _():
        pltpu.make_async_copy(den_v, den_sh.at[st[1]], semsd).wait()

      @plsc.parallel_loop(0, BB, unroll=8)
      def _(e):
        adrow = adv[e]
        z = asv[e] + adrow
        zm = mvec + adrow
        m = jnp.maximum(zm, 0.2 * zm)
        z = jnp.maximum(z, 0.2 * z) - m
        w = jnp.exp(z)
        den_v[e] = w
        for kk in range(4):
          bv = w.at[ivs[kk]].get(mode="promise_in_bounds")
          for t in range(2):
            k2 = 2 * kk + t
            out_v[e, pl.ds(16 * k2, 16)] = hv[e, pl.ds(16 * k2, 16)] * bv

      pltpu.async_copy(out_v, num_sh.at[st[1]], sems, add=True)

      @pl.when(c == 0)
      def _():
        pltpu.async_copy(den_v, den_sh.at[st[1]], semsd, add=True)

    idx_issue(0, seta)
    idx_wait(seta)
    gather_issue(seta)
    idx_issue(1, setb)

    @pl.loop(0, (NB - 2) // 2)
    def _(i):
      b0 = 2 * i
      for cur, nxt, boff in ((seta, setb, b0), (setb, seta, b0 + 1)):
        idx_wait(nxt)
        gather_issue(nxt)
        gather_wait(cur)
        compute_scatter(cur)
        idx_issue(boff + 2, cur)

    idx_wait(setb)
    gather_issue(setb)
    gather_wait(seta)
    compute_scatter(seta)
    gather_wait(setb)
    compute_scatter(setb)
    pltpu.make_async_copy(out_v, num_sh.at[setb[1]], sems).wait()

    @pl.when(c == 0)
    def _():
      pltpu.make_async_copy(den_v, den_sh.at[setb[1]], semsd).wait()

    plsc.subcore_barrier()
    pltpu.sync_copy(num_sh.at[pl.ds(r0, ROWS_PT)],
                    num_h.at[c, pl.ds(r0, ROWS_PT)])

    @pl.when(c == 0)
    def _():
      pltpu.sync_copy(den_sh.at[pl.ds(r0, ROWS_PT)],
                      den_h.at[pl.ds(r0, ROWS_PT)])

  return k(sidx2, dstv, asT2, adT2, hT, m16)


def _tc_pre_body(x_ref, w_ref, ps_ref, pd_ref, h2_ref, as2_ref, ad2_ref):
  h = lax.dot(x_ref[...], w_ref[...], preferred_element_type=F32)
  h2_ref[0] = h[:, :128]
  h2_ref[1] = h[:, 128:]
  lane = lax.broadcasted_iota(jnp.int32, (h.shape[0], 16), 1)
  a_s = lax.dot(h, ps_ref[...], precision=HIGH, preferred_element_type=F32)
  a_s = jnp.where(lane < 8, a_s, NEG)
  as2_ref[0] = a_s
  as2_ref[1] = a_s
  a_d = lax.dot(h, pd_ref[...], precision=HIGH, preferred_element_type=F32)
  a_d = jnp.where(lane < 8, a_d, NEG)
  ad2_ref[0] = a_d
  ad2_ref[1] = a_d


def _tc_pre(x, wg, ps, pd):
  n, d_in = x.shape
  rb = 1000
  return pl.pallas_call(
      _tc_pre_body,
      grid=(n // rb,),
      in_specs=[
          pl.BlockSpec((rb, d_in), lambda i: (i, 0)),
          pl.BlockSpec((d_in, HID), lambda i: (0, 0)),
          pl.BlockSpec((HID, 16), lambda i: (0, 0)),
          pl.BlockSpec((HID, 16), lambda i: (0, 0)),
      ],
      out_specs=[
          pl.BlockSpec((2, rb, 128), lambda i: (0, i, 0)),
          pl.BlockSpec((2, rb, 16), lambda i: (0, i, 0)),
          pl.BlockSpec((2, rb, 16), lambda i: (0, i, 0)),
      ],
      out_shape=[
          jax.ShapeDtypeStruct((2, NP, 128), F32),
          jax.ShapeDtypeStruct((2, NP, 16), F32),
          jax.ShapeDtypeStruct((2, NP, 16), F32),
      ],
  )(x, wg, ps, pd)


def _ln(v, sc, bi):
  mu = jnp.mean(v, axis=1, keepdims=True)
  var = jnp.mean((v - mu) ** 2, axis=1, keepdims=True)
  return (v - mu) / jnp.sqrt(var + 1e-5) * sc + bi


def _tc_post_body(first, num_ref, den_ref, xp_ref, r16_ref, bg_ref,
                  n1s_ref, n1b_ref, w1_ref, b1_ref, w2_ref, b2_ref,
                  n2s_ref, n2b_ref, o_ref):
  den_e = lax.dot(den_ref[...], r16_ref[...], precision=HIGH,
                  preferred_element_type=F32)
  gat = jnp.concatenate([num_ref[0], num_ref[1]], axis=1)
  gat = jnp.where(den_e > 0.0, gat / den_e, 0.0) + bg_ref[...]
  if first:
    t = gat
  else:
    t = xp_ref[...] + gat
  t = _ln(t, n1s_ref[...], n1b_ref[...])
  y = jnp.maximum(
      lax.dot(t, w1_ref[...], preferred_element_type=F32)
      + b1_ref[...], 0.0)
  y = lax.dot(y, w2_ref[...], preferred_element_type=F32) + b2_ref[...]
  o_ref[...] = _ln(t + y, n2s_ref[...], n2b_ref[...])


def _tc_post(first, num2, den, xprev, r16, bg, n1s, n1b, w1, b1, w2, b2,
             n2s, n2b):
  n = xprev.shape[0]
  rb = 1000
  dprev = xprev.shape[1]
  return pl.pallas_call(
      functools.partial(_tc_post_body, first),
      grid=(n // rb,),
      in_specs=[
          pl.BlockSpec((2, rb, 128), lambda i: (0, i, 0)),
          pl.BlockSpec((rb, 16), lambda i: (i, 0)),
          pl.BlockSpec((rb, dprev), lambda i: (i, 0)),
          pl.BlockSpec((16, HID), lambda i: (0, 0)),
          pl.BlockSpec((1, HID), lambda i: (0, 0)),
          pl.BlockSpec((1, HID), lambda i: (0, 0)),
          pl.BlockSpec((1, HID), lambda i: (0, 0)),
          pl.BlockSpec((HID, 2 * HID), lambda i: (0, 0)),
          pl.BlockSpec((1, 2 * HID), lambda i: (0, 0)),
          pl.BlockSpec((2 * HID, HID), lambda i: (0, 0)),
          pl.BlockSpec((1, HID), lambda i: (0, 0)),
          pl.BlockSpec((1, HID), lambda i: (0, 0)),
          pl.BlockSpec((1, HID), lambda i: (0, 0)),
      ],
      out_specs=pl.BlockSpec((rb, HID), lambda i: (i, 0)),
      out_shape=jax.ShapeDtypeStruct((n, HID), F32),
  )(num2, den, xprev, r16, bg, n1s, n1b, w1, b1, w2, b2, n2s, n2b)


def _tc_mid_body(first, num_ref, den_ref, xp_ref, r16_ref, bg_ref,
                 n1s_ref, n1b_ref, w1_ref, b1_ref, w2_ref, b2_ref,
                 n2s_ref, n2b_ref, wg_ref, ps_ref, pd_ref,
                 o_ref, h2_ref, as2_ref, ad2_ref):
  den_e = lax.dot(den_ref[...], r16_ref[...], precision=HIGH,
                  preferred_element_type=F32)
  gat = jnp.concatenate([num_ref[0], num_ref[1]], axis=1)
  gat = jnp.where(den_e > 0.0, gat / den_e, 0.0) + bg_ref[...]
  if first:
    t = gat
  else:
    t = xp_ref[...] + gat
  t = _ln(t, n1s_ref[...], n1b_ref[...])
  y = jnp.maximum(
      lax.dot(t, w1_ref[...], preferred_element_type=F32)
      + b1_ref[...], 0.0)
  y = lax.dot(y, w2_ref[...],
              preferred_element_type=F32) + b2_ref[...]
  xn = _ln(t + y, n2s_ref[...], n2b_ref[...])
  o_ref[...] = xn
  h = lax.dot(xn, wg_ref[...], preferred_element_type=F32)
  h2_ref[0] = h[:, :128]
  h2_ref[1] = h[:, 128:]
  lane = lax.broadcasted_iota(jnp.int32, (h.shape[0], 16), 1)
  a_s = lax.dot(h, ps_ref[...], precision=HIGH, preferred_element_type=F32)
  a_s = jnp.where(lane < 8, a_s, NEG)
  as2_ref[0] = a_s
  as2_ref[1] = a_s
  a_d = lax.dot(h, pd_ref[...], precision=HIGH, preferred_element_type=F32)
  a_d = jnp.where(lane < 8, a_d, NEG)
  ad2_ref[0] = a_d
  ad2_ref[1] = a_d


def _tc_mid(first, num2, den, xprev, r16, bg, n1s, n1b, w1, b1, w2, b2,
            n2s, n2b, wg, ps, pd):
  n = xprev.shape[0]
  rb = 1000
  dprev = xprev.shape[1]
  full = lambda shape: pl.BlockSpec(shape, lambda i: tuple(0 for _ in shape))
  return pl.pallas_call(
      functools.partial(_tc_mid_body, first),
      grid=(n // rb,),
      in_specs=[
          pl.BlockSpec((2, rb, 128), lambda i: (0, i, 0)),
          pl.BlockSpec((rb, 16), lambda i: (i, 0)),
          pl.BlockSpec((rb, dprev), lambda i: (i, 0)),
          full((16, HID)),
          full((1, HID)),
          full((1, HID)),
          full((1, HID)),
          full((HID, 2 * HID)),
          full((1, 2 * HID)),
          full((2 * HID, HID)),
          full((1, HID)),
          full((1, HID)),
          full((1, HID)),
          full((HID, HID)),
          full((HID, 16)),
          full((HID, 16)),
      ],
      out_specs=[
          pl.BlockSpec((rb, HID), lambda i: (i, 0)),
          pl.BlockSpec((2, rb, 128), lambda i: (0, i, 0)),
          pl.BlockSpec((2, rb, 16), lambda i: (0, i, 0)),
          pl.BlockSpec((2, rb, 16), lambda i: (0, i, 0)),
      ],
      out_shape=[
          jax.ShapeDtypeStruct((n, HID), F32),
          jax.ShapeDtypeStruct((2, NP, 128), F32),
          jax.ShapeDtypeStruct((2, NP, 16), F32),
          jax.ShapeDtypeStruct((2, NP, 16), F32),
      ],
  )(num2, den, xprev, r16, bg, n1s, n1b, w1, b1, w2, b2, n2s, n2b,
    wg, ps, pd)


def kernel(x, edge_index, params):
  src = edge_index[0]
  dst = edge_index[1]
  sidx2 = jnp.concatenate([src, src + NP])

  rows = jnp.arange(HID)
  r16 = (jnp.arange(16)[:, None] == (jnp.arange(HID)[None, :] // 32))
  r16 = r16.astype(F32)

  def mk_ps(p, key):
    return jnp.zeros((HID, 16), F32).at[rows, rows // 32].set(
        p[key].reshape(-1))

  p0 = params[0]
  h2, as2, ad2 = _tc_pre(x, p0["W_gat"], mk_ps(p0, "a_src"),
                         mk_ps(p0, "a_dst"))

  for li, p in enumerate(params):
    as_max = jnp.max(as2[0, :, :8], axis=0)
    m16 = jnp.concatenate([as_max, jnp.zeros((8,), F32)])

    num2, den = _sc_edge(
        sidx2, dst,
        as2.reshape(2 * NP, 16), ad2[0],
        h2.reshape(2 * NP, 128), m16)

    post_args = (num2, den, x, r16,
                 p["b_gat"].reshape(1, HID),
                 p["n1_s"].reshape(1, HID), p["n1_b"].reshape(1, HID),
                 p["W1"], p["b1"].reshape(1, 2 * HID),
                 p["W2"], p["b2"].reshape(1, HID),
                 p["n2_s"].reshape(1, HID), p["n2_b"].reshape(1, HID))
    if li + 1 < len(params):
      pn = params[li + 1]
      x, h2, as2, ad2 = _tc_mid(li == 0, *post_args, pn["W_gat"],
                                mk_ps(pn, "a_src"), mk_ps(pn, "a_dst"))
    else:
      x = _tc_post(li == 0, *post_args)
  return x


# speedup vs baseline: 101.0139x; 1.0298x over previous
import functools

import jax
import jax.numpy as jnp
from jax import lax
from jax.experimental import pallas as pl
from jax.experimental.pallas import tpu as pltpu
from jax.experimental.pallas import tpu_sc as plsc

NN = 10000
NP = 10240
EE = 320000
HEADS = 8
HID = 256
NC = 2
NS = 16
PER_TILE = EE // NS
BB = 80
NB = PER_TILE // BB
ROWS_PT = NP // NS
NEG = -1e30
F32 = jnp.float32
HIGH = jax.lax.Precision.HIGHEST


def _sc_edge(sidx2, dstv, asT2, adT2, hT, m16):
  mesh = plsc.VectorSubcoreMesh(core_axis_name="c", subcore_axis_name="s")

  @functools.partial(
      pl.kernel,
      out_type=(
          jax.ShapeDtypeStruct((NC, NP, 128), F32),
          jax.ShapeDtypeStruct((NP, 16), F32),
      ),
      mesh=mesh,
      compiler_params=pltpu.CompilerParams(use_tc_tiling_on_sc=False),
      scratch_types=[
          pltpu.VMEM((BB,), jnp.int32),
          pltpu.VMEM((BB,), jnp.int32),
          pltpu.VMEM((BB,), jnp.int32),
          pltpu.VMEM((BB,), jnp.int32),
          pltpu.VMEM((BB, 16), F32),
          pltpu.VMEM((BB, 16), F32),
          pltpu.VMEM((BB, 16), F32),
          pltpu.VMEM((BB, 16), F32),
          pltpu.VMEM((BB, 128), F32),
          pltpu.VMEM((BB, 128), F32),
          pltpu.VMEM((BB, 128), F32),
          pltpu.VMEM((BB, 16), F32),
          pltpu.VMEM((16,), F32),
          pltpu.VMEM_SHARED((NP, 128), F32),
          pltpu.VMEM_SHARED((NP, 16), F32),
      ] + [pltpu.SemaphoreType.DMA] * 12,
  )
  def k(sidx2_h, dst_h, as_h, ad_h, h_h, m_h, num_h, den_h,
        si_a, si_b, di_a, di_b, as_a, as_b, ad_a, ad_b, h_a, h_b,
        out_v, den_v, m_v, num_sh, den_sh,
        sas_a, sad_a, sh_a, ssi_a, sdi_a, sas_b, sad_b, sh_b, ssi_b, sdi_b,
        sems, semsd):
    c = lax.axis_index("c")
    s = lax.axis_index("s")
    zvec = jnp.zeros((16,), F32)

    @pl.loop(0, BB)
    def _(r):
      for k8 in range(8):
        out_v[r, pl.ds(16 * k8, 16)] = zvec
      den_v[r] = zvec

    r0 = s * ROWS_PT
    for j in range(8):
      pltpu.sync_copy(out_v, num_sh.at[pl.ds(r0 + BB * j, BB)])
      pltpu.sync_copy(den_v, den_sh.at[pl.ds(r0 + BB * j, BB)])
    izvec = jnp.zeros((16,), jnp.int32)
    for j in range(5):
      di_a[pl.ds(16 * j, 16)] = izvec
    pltpu.async_copy(out_v, num_sh.at[di_a], sems, add=True)

    @pl.when(c == 0)
    def _():
      pltpu.async_copy(den_v, den_sh.at[di_a], semsd, add=True)
    plsc.subcore_barrier()

    pltpu.sync_copy(m_h, m_v)
    mvec = m_v[...]
    ivs = [jnp.full((16,), 4 * c + kk, jnp.int32) for kk in range(4)]

    seta = (si_a, di_a, as_a, ad_a, h_a, sas_a, sad_a, sh_a, ssi_a, sdi_a)
    setb = (si_b, di_b, as_b, ad_b, h_b, sas_b, sad_b, sh_b, ssi_b, sdi_b)

    def idx_issue(bidx, st):
      base = s * PER_TILE + bidx * BB
      pltpu.async_copy(sidx2_h.at[pl.ds(c * EE + base, BB)], st[0], st[8])
      pltpu.async_copy(dst_h.at[pl.ds(base, BB)], st[1], st[9])

    def idx_wait(st):
      pltpu.make_async_copy(sidx2_h.at[pl.ds(0, BB)], st[0], st[8]).wait()
      pltpu.make_async_copy(dst_h.at[pl.ds(0, BB)], st[1], st[9]).wait()

    def gather_issue(st):
      pltpu.async_copy(as_h.at[st[0]], st[2], st[5])
      pltpu.async_copy(ad_h.at[st[1]], st[3], st[6])
      pltpu.async_copy(h_h.at[st[0]], st[4], st[7])

    def gather_wait(st):
      pltpu.make_async_copy(as_h.at[st[0]], st[2], st[5]).wait()
      pltpu.make_async_copy(ad_h.at[st[1]], st[3], st[6]).wait()
      pltpu.make_async_copy(h_h.at[st[0]], st[4], st[7]).wait()

    def compute_scatter(st):
      asv, adv, hv = st[2], st[3], st[4]
      pltpu.make_async_copy(out_v, num_sh.at[st[1]], sems).wait()

      @pl.when(c == 0)
      def _():
        pltpu.make_async_copy(den_v, den_sh.at[st[1]], semsd).wait()

      @plsc.parallel_loop(0, BB, unroll=16)
      def _(e):
        adrow = adv[e]
        z = asv[e] + adrow
        zm = mvec + adrow
        m = jnp.maximum(zm, 0.2 * zm)
        z = jnp.maximum(z, 0.2 * z) - m
        w = jnp.exp(z)
        den_v[e] = w
        for kk in range(4):
          bv = w.at[ivs[kk]].get(mode="promise_in_bounds")
          for t in range(2):
            k2 = 2 * kk + t
            out_v[e, pl.ds(16 * k2, 16)] = hv[e, pl.ds(16 * k2, 16)] * bv

      pltpu.async_copy(out_v, num_sh.at[st[1]], sems, add=True)

      @pl.when(c == 0)
      def _():
        pltpu.async_copy(den_v, den_sh.at[st[1]], semsd, add=True)

    idx_issue(0, seta)
    idx_wait(seta)
    gather_issue(seta)
    idx_issue(1, setb)

    @pl.loop(0, (NB - 2) // 2)
    def _(i):
      b0 = 2 * i
      for cur, nxt, boff in ((seta, setb, b0), (setb, seta, b0 + 1)):
        idx_wait(nxt)
        gather_issue(nxt)
        gather_wait(cur)
        compute_scatter(cur)
        idx_issue(boff + 2, cur)

    idx_wait(setb)
    gather_issue(setb)
    gather_wait(seta)
    compute_scatter(seta)
    gather_wait(setb)
    compute_scatter(setb)
    pltpu.make_async_copy(out_v, num_sh.at[setb[1]], sems).wait()

    @pl.when(c == 0)
    def _():
      pltpu.make_async_copy(den_v, den_sh.at[setb[1]], semsd).wait()

    plsc.subcore_barrier()
    pltpu.sync_copy(num_sh.at[pl.ds(r0, ROWS_PT)],
                    num_h.at[c, pl.ds(r0, ROWS_PT)])

    @pl.when(c == 0)
    def _():
      pltpu.sync_copy(den_sh.at[pl.ds(r0, ROWS_PT)],
                      den_h.at[pl.ds(r0, ROWS_PT)])

  return k(sidx2, dstv, asT2, adT2, hT, m16)


def _tc_pre_body(x_ref, w_ref, ps_ref, pd_ref, h2_ref, as2_ref, ad2_ref):
  h = lax.dot(x_ref[...], w_ref[...], preferred_element_type=F32)
  h2_ref[0] = h[:, :128]
  h2_ref[1] = h[:, 128:]
  lane = lax.broadcasted_iota(jnp.int32, (h.shape[0], 16), 1)
  a_s = lax.dot(h, ps_ref[...], precision=HIGH, preferred_element_type=F32)
  a_s = jnp.where(lane < 8, a_s, NEG)
  as2_ref[0] = a_s
  as2_ref[1] = a_s
  a_d = lax.dot(h, pd_ref[...], precision=HIGH, preferred_element_type=F32)
  a_d = jnp.where(lane < 8, a_d, NEG)
  ad2_ref[0] = a_d
  ad2_ref[1] = a_d


def _tc_pre(x, wg, ps, pd):
  n, d_in = x.shape
  rb = 1000
  return pl.pallas_call(
      _tc_pre_body,
      grid=(n // rb,),
      in_specs=[
          pl.BlockSpec((rb, d_in), lambda i: (i, 0)),
          pl.BlockSpec((d_in, HID), lambda i: (0, 0)),
          pl.BlockSpec((HID, 16), lambda i: (0, 0)),
          pl.BlockSpec((HID, 16), lambda i: (0, 0)),
      ],
      out_specs=[
          pl.BlockSpec((2, rb, 128), lambda i: (0, i, 0)),
          pl.BlockSpec((2, rb, 16), lambda i: (0, i, 0)),
          pl.BlockSpec((2, rb, 16), lambda i: (0, i, 0)),
      ],
      out_shape=[
          jax.ShapeDtypeStruct((2, NP, 128), F32),
          jax.ShapeDtypeStruct((2, NP, 16), F32),
          jax.ShapeDtypeStruct((2, NP, 16), F32),
      ],
  )(x, wg, ps, pd)


def _ln(v, sc, bi):
  mu = jnp.mean(v, axis=1, keepdims=True)
  var = jnp.mean((v - mu) ** 2, axis=1, keepdims=True)
  return (v - mu) / jnp.sqrt(var + 1e-5) * sc + bi


def _tc_post_body(first, num_ref, den_ref, xp_ref, r16_ref, bg_ref,
                  n1s_ref, n1b_ref, w1_ref, b1_ref, w2_ref, b2_ref,
                  n2s_ref, n2b_ref, o_ref):
  den_e = lax.dot(den_ref[...], r16_ref[...], precision=HIGH,
                  preferred_element_type=F32)
  gat = jnp.concatenate([num_ref[0], num_ref[1]], axis=1)
  gat = jnp.where(den_e > 0.0, gat / den_e, 0.0) + bg_ref[...]
  if first:
    t = gat
  else:
    t = xp_ref[...] + gat
  t = _ln(t, n1s_ref[...], n1b_ref[...])
  y = jnp.maximum(
      lax.dot(t, w1_ref[...], preferred_element_type=F32)
      + b1_ref[...], 0.0)
  y = lax.dot(y, w2_ref[...], preferred_element_type=F32) + b2_ref[...]
  o_ref[...] = _ln(t + y, n2s_ref[...], n2b_ref[...])


def _tc_post(first, num2, den, xprev, r16, bg, n1s, n1b, w1, b1, w2, b2,
             n2s, n2b):
  n = xprev.shape[0]
  rb = 1000
  dprev = xprev.shape[1]
  return pl.pallas_call(
      functools.partial(_tc_post_body, first),
      grid=(n // rb,),
      in_specs=[
          pl.BlockSpec((2, rb, 128), lambda i: (0, i, 0)),
          pl.BlockSpec((rb, 16), lambda i: (i, 0)),
          pl.BlockSpec((rb, dprev), lambda i: (i, 0)),
          pl.BlockSpec((16, HID), lambda i: (0, 0)),
          pl.BlockSpec((1, HID), lambda i: (0, 0)),
          pl.BlockSpec((1, HID), lambda i: (0, 0)),
          pl.BlockSpec((1, HID), lambda i: (0, 0)),
          pl.BlockSpec((HID, 2 * HID), lambda i: (0, 0)),
          pl.BlockSpec((1, 2 * HID), lambda i: (0, 0)),
          pl.BlockSpec((2 * HID, HID), lambda i: (0, 0)),
          pl.BlockSpec((1, HID), lambda i: (0, 0)),
          pl.BlockSpec((1, HID), lambda i: (0, 0)),
          pl.BlockSpec((1, HID), lambda i: (0, 0)),
      ],
      out_specs=pl.BlockSpec((rb, HID), lambda i: (i, 0)),
      out_shape=jax.ShapeDtypeStruct((n, HID), F32),
  )(num2, den, xprev, r16, bg, n1s, n1b, w1, b1, w2, b2, n2s, n2b)


def _tc_mid_body(first, num_ref, den_ref, xp_ref, r16_ref, bg_ref,
                 n1s_ref, n1b_ref, w1_ref, b1_ref, w2_ref, b2_ref,
                 n2s_ref, n2b_ref, wg_ref, ps_ref, pd_ref,
                 o_ref, h2_ref, as2_ref, ad2_ref):
  den_e = lax.dot(den_ref[...], r16_ref[...], precision=HIGH,
                  preferred_element_type=F32)
  gat = jnp.concatenate([num_ref[0], num_ref[1]], axis=1)
  gat = jnp.where(den_e > 0.0, gat / den_e, 0.0) + bg_ref[...]
  if first:
    t = gat
  else:
    t = xp_ref[...] + gat
  t = _ln(t, n1s_ref[...], n1b_ref[...])
  y = jnp.maximum(
      lax.dot(t, w1_ref[...], preferred_element_type=F32)
      + b1_ref[...], 0.0)
  y = lax.dot(y, w2_ref[...],
              preferred_element_type=F32) + b2_ref[...]
  xn = _ln(t + y, n2s_ref[...], n2b_ref[...])
  o_ref[...] = xn
  h = lax.dot(xn, wg_ref[...], preferred_element_type=F32)
  h2_ref[0] = h[:, :128]
  h2_ref[1] = h[:, 128:]
  lane = lax.broadcasted_iota(jnp.int32, (h.shape[0], 16), 1)
  a_s = lax.dot(h, ps_ref[...], precision=HIGH, preferred_element_type=F32)
  a_s = jnp.where(lane < 8, a_s, NEG)
  as2_ref[0] = a_s
  as2_ref[1] = a_s
  a_d = lax.dot(h, pd_ref[...], precision=HIGH, preferred_element_type=F32)
  a_d = jnp.where(lane < 8, a_d, NEG)
  ad2_ref[0] = a_d
  ad2_ref[1] = a_d


def _tc_mid(first, num2, den, xprev, r16, bg, n1s, n1b, w1, b1, w2, b2,
            n2s, n2b, wg, ps, pd):
  n = xprev.shape[0]
  rb = 1000
  dprev = xprev.shape[1]
  full = lambda shape: pl.BlockSpec(shape, lambda i: tuple(0 for _ in shape))
  return pl.pallas_call(
      functools.partial(_tc_mid_body, first),
      grid=(n // rb,),
      in_specs=[
          pl.BlockSpec((2, rb, 128), lambda i: (0, i, 0)),
          pl.BlockSpec((rb, 16), lambda i: (i, 0)),
          pl.BlockSpec((rb, dprev), lambda i: (i, 0)),
          full((16, HID)),
          full((1, HID)),
          full((1, HID)),
          full((1, HID)),
          full((HID, 2 * HID)),
          full((1, 2 * HID)),
          full((2 * HID, HID)),
          full((1, HID)),
          full((1, HID)),
          full((1, HID)),
          full((HID, HID)),
          full((HID, 16)),
          full((HID, 16)),
      ],
      out_specs=[
          pl.BlockSpec((rb, HID), lambda i: (i, 0)),
          pl.BlockSpec((2, rb, 128), lambda i: (0, i, 0)),
          pl.BlockSpec((2, rb, 16), lambda i: (0, i, 0)),
          pl.BlockSpec((2, rb, 16), lambda i: (0, i, 0)),
      ],
      out_shape=[
          jax.ShapeDtypeStruct((n, HID), F32),
          jax.ShapeDtypeStruct((2, NP, 128), F32),
          jax.ShapeDtypeStruct((2, NP, 16), F32),
          jax.ShapeDtypeStruct((2, NP, 16), F32),
      ],
  )(num2, den, xprev, r16, bg, n1s, n1b, w1, b1, w2, b2, n2s, n2b,
    wg, ps, pd)


def kernel(x, edge_index, params):
  src = edge_index[0]
  dst = edge_index[1]
  sidx2 = jnp.concatenate([src, src + NP])

  rows = jnp.arange(HID)
  r16 = (jnp.arange(16)[:, None] == (jnp.arange(HID)[None, :] // 32))
  r16 = r16.astype(F32)

  def mk_ps(p, key):
    return jnp.zeros((HID, 16), F32).at[rows, rows // 32].set(
        p[key].reshape(-1))

  p0 = params[0]
  h2, as2, ad2 = _tc_pre(x, p0["W_gat"], mk_ps(p0, "a_src"),
                         mk_ps(p0, "a_dst"))

  for li, p in enumerate(params):
    as_max = jnp.max(as2[0, :, :8], axis=0)
    m16 = jnp.concatenate([as_max, jnp.zeros((8,), F32)])

    num2, den = _sc_edge(
        sidx2, dst,
        as2.reshape(2 * NP, 16), ad2[0],
        h2.reshape(2 * NP, 128), m16)

    post_args = (num2, den, x, r16,
                 p["b_gat"].reshape(1, HID),
                 p["n1_s"].reshape(1, HID), p["n1_b"].reshape(1, HID),
                 p["W1"], p["b1"].reshape(1, 2 * HID),
                 p["W2"], p["b2"].reshape(1, HID),
                 p["n2_s"].reshape(1, HID), p["n2_b"].reshape(1, HID))
    if li + 1 < len(params):
      pn = params[li + 1]
      x, h2, as2, ad2 = _tc_mid(li == 0, *post_args, pn["W_gat"],
                                mk_ps(pn, "a_src"), mk_ps(pn, "a_dst"))
    else:
      x = _tc_post(li == 0, *post_args)
  return x
